# Initial kernel scaffold; baseline (speedup 1.0000x reference)
import jax
import jax.numpy as jnp
from jax.experimental import pallas as pl


def kernel(inputs, z):
    raise NotImplementedError("write your pallas kernel here")


# speedup vs baseline: 1.3262x; 1.3262x over previous
import functools

import jax
import jax.numpy as jnp
from jax import lax
from jax.experimental import pallas as pl
from jax.experimental.pallas import tpu as pltpu
from jax.experimental.pallas import tpu_sc as plsc

NC, NS, LANES = 2, 16, 16
NW = NC * NS

B, H, D = 4096, 50, 128
BPW = B // NW
G = 4
NG = BPW // G
ROWS = G * H
OUT_W = D * H

_mesh = plsc.VectorSubcoreMesh(
    core_axis_name="c", subcore_axis_name="s", num_cores=NC, num_subcores=NS
)


@functools.partial(
    pl.kernel,
    out_type=jax.ShapeDtypeStruct((B * D * H,), jnp.float32),
    mesh=_mesh,
    compiler_params=pltpu.CompilerParams(needs_layout_passes=False),
    scratch_types=[
        pltpu.VMEM((BPW * H,), jnp.int32),
        pltpu.VMEM((ROWS, D), jnp.float32),
        pltpu.VMEM((G * OUT_W,), jnp.float32),
        pltpu.SemaphoreType.DMA,
    ],
)
def _gather_transpose(idx_hbm, z_hbm, out_hbm, idx_v, in_v, out_v, sem):
    wid = lax.axis_index("s") * NC + lax.axis_index("c")
    ibase = wid * (BPW * H)
    pltpu.sync_copy(idx_hbm.at[pl.ds(ibase, BPW * H)], idx_v)

    lane_off = lax.iota(jnp.int32, LANES) * H

    def group_body(gi, carry):
        pltpu.async_copy(
            z_hbm.at[idx_v.at[pl.ds(gi * ROWS, ROWS)]], in_v, sem
        ).wait()

        def t_body(t, c):
            for g in range(G):
                r = g * H + t
                obase = g * OUT_W + t
                for db in range(D // LANES):
                    v = in_v[r, pl.ds(db * LANES, LANES)]
                    idxv = lane_off + (obase + db * LANES * H)
                    plsc.store_scatter(out_v, [idxv], v)
            return c

        lax.fori_loop(0, H, t_body, 0)
        pltpu.sync_copy(
            out_v,
            out_hbm.at[pl.ds((wid * BPW + gi * G) * OUT_W, G * OUT_W)],
        )
        return carry

    lax.fori_loop(0, NG, group_body, 0)


def kernel(inputs, z):
    idx = inputs.reshape(-1).astype(jnp.int32)
    out = _gather_transpose(idx, z)
    return out.reshape(B, D, H)


# speedup vs baseline: 1.5050x; 1.1348x over previous
import functools

import jax
import jax.numpy as jnp
from jax import lax
from jax.experimental import pallas as pl
from jax.experimental.pallas import tpu as pltpu
from jax.experimental.pallas import tpu_sc as plsc

NC, NS, LANES = 2, 16, 16
NW = NC * NS

B, H, D = 4096, 50, 128
BPW = B // NW
G = 4
NG = BPW // G
ROWS = G * H
OUT_W = D * H

_mesh = plsc.VectorSubcoreMesh(
    core_axis_name="c", subcore_axis_name="s", num_cores=NC, num_subcores=NS
)


@functools.partial(
    pl.kernel,
    out_type=jax.ShapeDtypeStruct((B * D * H,), jnp.float32),
    mesh=_mesh,
    compiler_params=pltpu.CompilerParams(needs_layout_passes=False),
    scratch_types=[
        pltpu.VMEM((BPW * H,), jnp.int32),
        pltpu.VMEM((2, ROWS, D), jnp.float32),
        pltpu.VMEM((2 * G * OUT_W,), jnp.float32),
        pltpu.SemaphoreType.DMA,
        pltpu.SemaphoreType.DMA,
        pltpu.SemaphoreType.DMA,
        pltpu.SemaphoreType.DMA,
    ],
)
def _gather_transpose(
    idx_hbm, z_hbm, out_hbm, idx_v, in_v, out_v, si0, si1, so0, so1
):
    sem_i = (si0, si1)
    sem_o = (so0, so1)
    wid = lax.axis_index("s") * NC + lax.axis_index("c")
    ibase = wid * (BPW * H)
    obase = wid * (BPW * OUT_W)
    pltpu.sync_copy(idx_hbm.at[pl.ds(ibase, BPW * H)], idx_v)

    lane_off = lax.iota(jnp.int32, LANES) * H

    def gather_start(gi, b):
        pltpu.make_async_copy(
            z_hbm.at[idx_v.at[pl.ds(gi * ROWS, ROWS)]], in_v.at[b], sem_i[b]
        ).start()

    def gather_wait(b):
        pltpu.make_async_copy(
            z_hbm.at[idx_v.at[pl.ds(0, ROWS)]], in_v.at[b], sem_i[b]
        ).wait()

    def out_start(gi, b):
        pltpu.make_async_copy(
            out_v.at[pl.ds(b * G * OUT_W, G * OUT_W)],
            out_hbm.at[pl.ds(obase + gi * (G * OUT_W), G * OUT_W)],
            sem_o[b],
        ).start()

    def out_wait(b):
        pltpu.make_async_copy(
            out_v.at[pl.ds(b * G * OUT_W, G * OUT_W)],
            out_hbm.at[pl.ds(0, G * OUT_W)],
            sem_o[b],
        ).wait()

    gather_start(0, 0)
    gather_start(1, 1)

    def pair_body(i, carry):
        for b in range(2):
            gi = i * 2 + b
            gather_wait(b)

            @pl.when(gi >= 2)
            def _():
                out_wait(b)

            def t_body(t, c):
                for g in range(G):
                    r = g * H + t
                    ob = (b * G + g) * OUT_W + t
                    for db in range(D // LANES):
                        v = in_v[b, r, pl.ds(db * LANES, LANES)]
                        idxv = lane_off + (ob + db * LANES * H)
                        plsc.store_scatter(out_v, [idxv], v)
                return c

            lax.fori_loop(0, H, t_body, 0)
            out_start(gi, b)

            @pl.when(gi + 2 < NG)
            def _():
                gather_start(gi + 2, b)
        return carry

    lax.fori_loop(0, NG // 2, pair_body, 0)
    out_wait(0)
    out_wait(1)


def kernel(inputs, z):
    idx = inputs.reshape(-1).astype(jnp.int32)
    out = _gather_transpose(idx, z)
    return out.reshape(B, D, H)


# speedup vs baseline: 10.8776x; 7.2278x over previous
import functools

import jax
import jax.numpy as jnp
from jax import lax
from jax.experimental import pallas as pl
from jax.experimental.pallas import tpu as pltpu
from jax.experimental.pallas import tpu_sc as plsc

NC, NS = 2, 16
NW = NC * NS

B, H, D = 4096, 50, 128
ROWS = B * H
RPW = ROWS // NW
CHUNK = 128
NCH = RPW // CHUNK
NBUF = 5

_mesh = plsc.VectorSubcoreMesh(
    core_axis_name="c", subcore_axis_name="s", num_cores=NC, num_subcores=NS
)


@functools.partial(
    pl.kernel,
    out_type=jax.ShapeDtypeStruct((ROWS, D), jnp.float32),
    mesh=_mesh,
    compiler_params=pltpu.CompilerParams(needs_layout_passes=False),
    scratch_types=[
        pltpu.VMEM((RPW,), jnp.int32),
        pltpu.VMEM((NBUF, CHUNK, D), jnp.float32),
        [pltpu.SemaphoreType.DMA] * NBUF,
        [pltpu.SemaphoreType.DMA] * NBUF,
    ],
)
def _row_gather(tidx_hbm, z_hbm, out_hbm, idx_v, in_v, sem_g, sem_o):
    wid = lax.axis_index("s") * NC + lax.axis_index("c")
    jbase = wid * RPW
    pltpu.sync_copy(tidx_hbm.at[pl.ds(jbase, RPW)], idx_v)

    def gather_start(ci, b):
        pltpu.make_async_copy(
            z_hbm.at[idx_v.at[pl.ds(ci * CHUNK, CHUNK)]], in_v.at[b], sem_g[b]
        ).start()

    def gather_wait(b):
        pltpu.make_async_copy(
            z_hbm.at[idx_v.at[pl.ds(0, CHUNK)]], in_v.at[b], sem_g[b]
        ).wait()

    def out_start(ci, b):
        pltpu.make_async_copy(
            in_v.at[b],
            out_hbm.at[pl.ds(jbase + ci * CHUNK, CHUNK), :],
            sem_o[b],
        ).start()

    def out_wait(b):
        pltpu.make_async_copy(
            in_v.at[b], out_hbm.at[pl.ds(0, CHUNK), :], sem_o[b]
        ).wait()

    for b in range(NBUF):
        gather_start(b, b)

    def ring_body(i, carry):
        for b in range(NBUF):
            ci = i * NBUF + b
            gather_wait(b)
            out_start(ci, b)

            @pl.when(ci + NBUF < NCH)
            def _():
                out_wait(b)
                gather_start(ci + NBUF, b)

        return carry

    lax.fori_loop(0, NCH // NBUF, ring_body, 0)
    for b in range(NBUF):
        out_wait(b)


def kernel(inputs, z):
    tidx = inputs.T.reshape(-1).astype(jnp.int32)
    rows = _row_gather(tidx, z)
    return rows.reshape(H, B, D).transpose(1, 2, 0)
